# baseline (speedup 1.0000x reference)
import functools

import jax
import jax.numpy as jnp
from jax import lax
from jax.experimental import pallas as pl
from jax.experimental.pallas import tpu as pltpu
from jax.experimental.pallas import tpu_sc as plsc

N = 10000
E = 320000
F = 128
HID2 = 127
CF = 64
OF = 16
SLOPE = (1.0 / 8.0 + 1.0 / 3.0) / 2.0

NC = 2
NS = 16
NW = NC * NS
EPW = E // NW
K = 80
NCHUNK = EPW // K
NP = 10240
RPT = NP // NS

_mesh = plsc.VectorSubcoreMesh(core_axis_name="c", subcore_axis_name="s")


@functools.partial(
    pl.kernel,
    out_type=jax.ShapeDtypeStruct((NC, NP, F), jnp.float32),
    mesh=_mesh,
    scratch_types=[
        pltpu.VMEM((NCHUNK, K), jnp.int32),
        pltpu.VMEM((K, F), jnp.float32),
        pltpu.VMEM_SHARED((NP, F), jnp.float32),
    ],
)
def _deg_kernel(dst_hbm, ones_hbm, zeros_hbm, out_hbm, dst_v, ones_v, acc):
    c = lax.axis_index("c")
    s = lax.axis_index("s")
    w = c * NS + s
    pltpu.sync_copy(dst_hbm.at[w], dst_v)
    pltpu.sync_copy(ones_hbm, ones_v)
    pltpu.sync_copy(zeros_hbm.at[pl.ds(s * RPT, RPT)], acc.at[pl.ds(s * RPT, RPT)])
    plsc.subcore_barrier()

    def body(j, carry):
        pltpu.sync_copy(ones_v, acc.at[dst_v.at[j]], add=True)
        return carry

    lax.fori_loop(0, NCHUNK, body, 0)
    plsc.subcore_barrier()
    pltpu.sync_copy(acc.at[pl.ds(s * RPT, RPT)], out_hbm.at[c, pl.ds(s * RPT, RPT)])


@functools.partial(
    pl.kernel,
    out_type=jax.ShapeDtypeStruct((NC, NP, F), jnp.float32),
    mesh=_mesh,
    scratch_types=[
        pltpu.VMEM((NCHUNK, K), jnp.int32),
        pltpu.VMEM((NCHUNK, K), jnp.int32),
        pltpu.VMEM((K, F), jnp.float32),
        pltpu.VMEM_SHARED((NP, F), jnp.float32),
    ],
)
def _agg_kernel(x_hbm, src_hbm, dst_hbm, zeros_hbm, out_hbm,
                src_v, dst_v, rows_v, acc):
    c = lax.axis_index("c")
    s = lax.axis_index("s")
    w = c * NS + s
    pltpu.sync_copy(src_hbm.at[w], src_v)
    pltpu.sync_copy(dst_hbm.at[w], dst_v)
    pltpu.sync_copy(zeros_hbm.at[pl.ds(s * RPT, RPT)], acc.at[pl.ds(s * RPT, RPT)])
    plsc.subcore_barrier()

    def body(j, carry):
        pltpu.sync_copy(x_hbm.at[src_v.at[j]], rows_v)
        pltpu.sync_copy(rows_v, acc.at[dst_v.at[j]], add=True)
        return carry

    lax.fori_loop(0, NCHUNK, body, 0)
    plsc.subcore_barrier()
    pltpu.sync_copy(acc.at[pl.ds(s * RPT, RPT)], out_hbm.at[c, pl.ds(s * RPT, RPT)])


def _gru(x, Wih, Whh, bih, bhh, d):
    gi = jnp.dot(x, Wih, preferred_element_type=jnp.float32) + bih
    gh = jnp.dot(x, Whh, preferred_element_type=jnp.float32) + bhh
    i_r, i_z, i_n = gi[:, :d], gi[:, d:2 * d], gi[:, 2 * d:]
    h_r, h_z, h_n = gh[:, :d], gh[:, d:2 * d], gh[:, 2 * d:]
    r = jax.nn.sigmoid(i_r + h_r)
    z = jax.nn.sigmoid(i_z + h_z)
    n = jnp.tanh(i_n + r * h_n)
    return (1.0 - z) * n + z * x


def _evolve_body(ns_ref, W1_ref, W2_ref, Wih1_ref, Whh1_ref, bih1_ref, bhh1_ref,
                 Wih2_ref, Whh2_ref, bih2_ref, bhh2_ref, Wm1_ref,
                 W1f_ref, Wc_ref):
    ns = ns_ref[0]

    def body(t, carry):
        W1, W2 = carry
        W1 = _gru(W1, Wih1_ref[...], Whh1_ref[...], bih1_ref[...], bhh1_ref[...], F)
        W2 = _gru(W2, Wih2_ref[...], Whh2_ref[...], bih2_ref[...], bhh2_ref[...], HID2)
        return (W1, W2)

    W1f, W2f = lax.fori_loop(0, ns, body, (W1_ref[...], W2_ref[...]))
    W1f_ref[...] = W1f
    Wc_ref[...] = jnp.dot(W2f, Wm1_ref[...], preferred_element_type=jnp.float32)


def _prep_body(d0_ref, d1_ref, x_ref, xp_ref, dinv_ref):
    deg = d0_ref[:, 0:1] + d1_ref[:, 0:1]
    dinv = lax.rsqrt(jnp.maximum(deg, 1.0))
    dinv_ref[...] = dinv
    xp_ref[...] = x_ref[...] * dinv


def _mid_body(z0_ref, z1_ref, dinv_ref, W1f_ref, hp_ref):
    dinv = dinv_ref[...]
    z = (z0_ref[...] + z1_ref[...]) * dinv
    h = jnp.dot(z, W1f_ref[...], preferred_element_type=jnp.float32)
    h = jnp.where(h >= 0.0, h, SLOPE * h)
    hp_ref[...] = h * dinv


def _final_body(u0_ref, u1_ref, dinv_ref, Wc_ref, bm1_ref, Wm2_ref, bm2_ref, out_ref):
    u = (u0_ref[...] + u1_ref[...]) * dinv_ref[...]
    t = jnp.dot(u, Wc_ref[...], preferred_element_type=jnp.float32) + bm1_ref[...]
    t = jnp.maximum(t, 0.0)
    out_ref[...] = jnp.dot(t, Wm2_ref[...], preferred_element_type=jnp.float32) + bm2_ref[...]


def kernel(feat_list, edge_index, n_step, W1, W2, Wih1, Whh1, bih1, bhh1,
           Wih2, Whh2, bih2, bhh2, Wm1, bm1, Wm2, bm2):
    src = edge_index[0].reshape(NW, NCHUNK, K)
    dst = edge_index[1].reshape(NW, NCHUNK, K)
    x_last = lax.dynamic_index_in_dim(feat_list, n_step - 1, 0, keepdims=False)
    x_last = jnp.pad(x_last, ((0, NP - N), (0, 0)))

    ones_d = jnp.ones((K, F), jnp.float32)
    zeros_f = jnp.zeros((NP, F), jnp.float32)

    deg_parts = _deg_kernel(dst, ones_d, zeros_f)

    W1f, Wc = pl.pallas_call(
        _evolve_body,
        out_shape=[
            jax.ShapeDtypeStruct((F, F), jnp.float32),
            jax.ShapeDtypeStruct((F, CF), jnp.float32),
        ],
        in_specs=[pl.BlockSpec(memory_space=pltpu.SMEM)] + [pl.BlockSpec()] * 11,
    )(jnp.asarray(n_step, jnp.int32).reshape(1), W1, W2,
      Wih1, Whh1, bih1.reshape(1, -1), bhh1.reshape(1, -1),
      Wih2, Whh2, bih2.reshape(1, -1), bhh2.reshape(1, -1), Wm1)

    xp, dinv = pl.pallas_call(
        _prep_body,
        out_shape=[
            jax.ShapeDtypeStruct((NP, F), jnp.float32),
            jax.ShapeDtypeStruct((NP, 1), jnp.float32),
        ],
    )(deg_parts[0], deg_parts[1], x_last)

    z_parts = _agg_kernel(xp, src, dst, zeros_f)

    hp = pl.pallas_call(
        _mid_body,
        out_shape=jax.ShapeDtypeStruct((NP, F), jnp.float32),
    )(z_parts[0], z_parts[1], dinv, W1f)

    u_parts = _agg_kernel(hp, src, dst, zeros_f)

    out = pl.pallas_call(
        _final_body,
        out_shape=jax.ShapeDtypeStruct((NP, OF), jnp.float32),
    )(u_parts[0], u_parts[1], dinv, Wc, bm1.reshape(1, -1), Wm2, bm2.reshape(1, -1))
    return out[:N]


# speedup vs baseline: 1.4391x; 1.4391x over previous
import functools

import jax
import jax.numpy as jnp
from jax import lax
from jax.experimental import pallas as pl
from jax.experimental.pallas import tpu as pltpu
from jax.experimental.pallas import tpu_sc as plsc

N = 10000
E = 320000
F = 128
HID2 = 127
CF = 64
OF = 16
SLOPE = (1.0 / 8.0 + 1.0 / 3.0) / 2.0

NC = 2
NS = 16
NW = NC * NS
EPW = E // NW
K = 80
NCHUNK = EPW // K
NP = 10240
RPT = NP // NS

_mesh = plsc.VectorSubcoreMesh(core_axis_name="c", subcore_axis_name="s")


@functools.partial(
    pl.kernel,
    out_type=jax.ShapeDtypeStruct((NC, NP, F), jnp.float32),
    mesh=_mesh,
    scratch_types=[
        pltpu.VMEM((EPW,), jnp.int32),
        pltpu.VMEM((K, F), jnp.float32),
        pltpu.VMEM_SHARED((NP, F), jnp.float32),
    ],
)
def _deg_kernel(dst_hbm, ones_hbm, zeros_hbm, out_hbm, dst_v, ones_v, acc):
    c = lax.axis_index("c")
    s = lax.axis_index("s")
    w = c * NS + s
    pltpu.sync_copy(dst_hbm.at[w], dst_v)
    pltpu.sync_copy(ones_hbm, ones_v)
    pltpu.sync_copy(zeros_hbm.at[pl.ds(s * RPT, RPT)], acc.at[pl.ds(s * RPT, RPT)])
    plsc.subcore_barrier()

    def body(j, carry):
        pltpu.sync_copy(ones_v, acc.at[dst_v.at[pl.ds(j * K, K)]], add=True)
        return carry

    lax.fori_loop(0, NCHUNK, body, 0)
    plsc.subcore_barrier()
    pltpu.sync_copy(acc.at[pl.ds(s * RPT, RPT)], out_hbm.at[c, pl.ds(s * RPT, RPT)])


@functools.partial(
    pl.kernel,
    out_type=jax.ShapeDtypeStruct((NC, NP, F), jnp.float32),
    mesh=_mesh,
    scratch_types=[
        pltpu.VMEM((EPW,), jnp.int32),
        pltpu.VMEM((EPW,), jnp.int32),
        pltpu.VMEM((2, K, F), jnp.float32),
        pltpu.VMEM_SHARED((NP, F), jnp.float32),
        pltpu.SemaphoreType.DMA,
        pltpu.SemaphoreType.DMA,
    ],
)
def _agg_kernel(x_hbm, src_hbm, dst_hbm, zeros_hbm, out_hbm,
                src_v, dst_v, rows_v, acc, sem0, sem1):
    c = lax.axis_index("c")
    s = lax.axis_index("s")
    w = c * NS + s
    pltpu.sync_copy(src_hbm.at[w], src_v)
    pltpu.sync_copy(dst_hbm.at[w], dst_v)
    pltpu.sync_copy(zeros_hbm.at[pl.ds(s * RPT, RPT)], acc.at[pl.ds(s * RPT, RPT)])
    plsc.subcore_barrier()

    sems = (sem0, sem1)
    for b in range(2):
        pltpu.async_copy(x_hbm.at[src_v.at[pl.ds(b * K, K)]], rows_v.at[b],
                         sems[b])

    def body(gi, carry):
        g = gi * 2
        for b in range(2):
            j = g + b
            pltpu.make_async_copy(x_hbm.at[src_v.at[pl.ds(j * K, K)]],
                                  rows_v.at[b], sems[b]).wait()
            pltpu.sync_copy(rows_v.at[b], acc.at[dst_v.at[pl.ds(j * K, K)]],
                            add=True)
            pltpu.async_copy(x_hbm.at[src_v.at[pl.ds((j + 2) * K, K)]],
                             rows_v.at[b], sems[b])
        return carry

    lax.fori_loop(0, (NCHUNK - 3) // 2, body, 0)
    for b in range(2):
        j = NCHUNK - 3 + b
        pltpu.make_async_copy(x_hbm.at[src_v.at[pl.ds(j * K, K)]],
                              rows_v.at[b], sems[b]).wait()
        pltpu.sync_copy(rows_v.at[b], acc.at[dst_v.at[pl.ds(j * K, K)]],
                        add=True)
    j = NCHUNK - 1
    pltpu.sync_copy(x_hbm.at[src_v.at[pl.ds(j * K, K)]], rows_v.at[0])
    pltpu.sync_copy(rows_v.at[0], acc.at[dst_v.at[pl.ds(j * K, K)]], add=True)
    plsc.subcore_barrier()
    pltpu.sync_copy(acc.at[pl.ds(s * RPT, RPT)], out_hbm.at[c, pl.ds(s * RPT, RPT)])


def _gru(x, Wih, Whh, bih, bhh, d):
    gi = jnp.dot(x, Wih, preferred_element_type=jnp.float32) + bih
    gh = jnp.dot(x, Whh, preferred_element_type=jnp.float32) + bhh
    i_r, i_z, i_n = gi[:, :d], gi[:, d:2 * d], gi[:, 2 * d:]
    h_r, h_z, h_n = gh[:, :d], gh[:, d:2 * d], gh[:, 2 * d:]
    r = jax.nn.sigmoid(i_r + h_r)
    z = jax.nn.sigmoid(i_z + h_z)
    n = jnp.tanh(i_n + r * h_n)
    return (1.0 - z) * n + z * x


def _evolve_body(ns_ref, W1_ref, W2_ref, Wih1_ref, Whh1_ref, bih1_ref, bhh1_ref,
                 Wih2_ref, Whh2_ref, bih2_ref, bhh2_ref, Wm1_ref,
                 W1f_ref, Wc_ref):
    ns = ns_ref[0]

    def body(t, carry):
        W1, W2 = carry
        W1 = _gru(W1, Wih1_ref[...], Whh1_ref[...], bih1_ref[...], bhh1_ref[...], F)
        W2 = _gru(W2, Wih2_ref[...], Whh2_ref[...], bih2_ref[...], bhh2_ref[...], HID2)
        return (W1, W2)

    W1f, W2f = lax.fori_loop(0, ns, body, (W1_ref[...], W2_ref[...]))
    W1f_ref[...] = W1f
    Wc_ref[...] = jnp.dot(W2f, Wm1_ref[...], preferred_element_type=jnp.float32)


def _prep_body(d0_ref, d1_ref, x_ref, xp_ref, dinv_ref):
    deg = d0_ref[:, 0:1] + d1_ref[:, 0:1]
    dinv = lax.rsqrt(jnp.maximum(deg, 1.0))
    dinv_ref[...] = dinv
    xp_ref[...] = x_ref[...] * dinv


def _mid_body(z0_ref, z1_ref, dinv_ref, W1f_ref, hp_ref):
    dinv = dinv_ref[...]
    z = (z0_ref[...] + z1_ref[...]) * dinv
    h = jnp.dot(z, W1f_ref[...], preferred_element_type=jnp.float32)
    h = jnp.where(h >= 0.0, h, SLOPE * h)
    hp_ref[...] = h * dinv


def _final_body(u0_ref, u1_ref, dinv_ref, Wc_ref, bm1_ref, Wm2_ref, bm2_ref, out_ref):
    u = (u0_ref[...] + u1_ref[...]) * dinv_ref[...]
    t = jnp.dot(u, Wc_ref[...], preferred_element_type=jnp.float32) + bm1_ref[...]
    t = jnp.maximum(t, 0.0)
    out_ref[...] = jnp.dot(t, Wm2_ref[...], preferred_element_type=jnp.float32) + bm2_ref[...]


def kernel(feat_list, edge_index, n_step, W1, W2, Wih1, Whh1, bih1, bhh1,
           Wih2, Whh2, bih2, bhh2, Wm1, bm1, Wm2, bm2):
    src = edge_index[0].reshape(NW, EPW)
    dst = edge_index[1].reshape(NW, EPW)
    x_last = lax.dynamic_index_in_dim(feat_list, n_step - 1, 0, keepdims=False)
    x_last = jnp.pad(x_last, ((0, NP - N), (0, 0)))

    ones_d = jnp.ones((K, F), jnp.float32)
    zeros_f = jnp.zeros((NP, F), jnp.float32)

    deg_parts = _deg_kernel(dst, ones_d, zeros_f)

    W1f, Wc = pl.pallas_call(
        _evolve_body,
        out_shape=[
            jax.ShapeDtypeStruct((F, F), jnp.float32),
            jax.ShapeDtypeStruct((F, CF), jnp.float32),
        ],
        in_specs=[pl.BlockSpec(memory_space=pltpu.SMEM)] + [pl.BlockSpec()] * 11,
    )(jnp.asarray(n_step, jnp.int32).reshape(1), W1, W2,
      Wih1, Whh1, bih1.reshape(1, -1), bhh1.reshape(1, -1),
      Wih2, Whh2, bih2.reshape(1, -1), bhh2.reshape(1, -1), Wm1)

    xp, dinv = pl.pallas_call(
        _prep_body,
        out_shape=[
            jax.ShapeDtypeStruct((NP, F), jnp.float32),
            jax.ShapeDtypeStruct((NP, 1), jnp.float32),
        ],
    )(deg_parts[0], deg_parts[1], x_last)

    z_parts = _agg_kernel(xp, src, dst, zeros_f)

    hp = pl.pallas_call(
        _mid_body,
        out_shape=jax.ShapeDtypeStruct((NP, F), jnp.float32),
    )(z_parts[0], z_parts[1], dinv, W1f)

    u_parts = _agg_kernel(hp, src, dst, zeros_f)

    out = pl.pallas_call(
        _final_body,
        out_shape=jax.ShapeDtypeStruct((NP, OF), jnp.float32),
    )(u_parts[0], u_parts[1], dinv, Wc, bm1.reshape(1, -1), Wm2, bm2.reshape(1, -1))
    return out[:N]


# speedup vs baseline: 1.4905x; 1.0357x over previous
import functools

import jax
import jax.numpy as jnp
from jax import lax
from jax.experimental import pallas as pl
from jax.experimental.pallas import tpu as pltpu
from jax.experimental.pallas import tpu_sc as plsc

N = 10000
E = 320000
F = 128
HID2 = 127
CF = 64
OF = 16
SLOPE = (1.0 / 8.0 + 1.0 / 3.0) / 2.0

NC = 2
NS = 16
NW = NC * NS
EPW = E // NW
K = 40
NCHUNK = EPW // K
U = 5
NP = 10240
RPT = NP // NS

_mesh = plsc.VectorSubcoreMesh(core_axis_name="c", subcore_axis_name="s")


@functools.partial(
    pl.kernel,
    out_type=jax.ShapeDtypeStruct((NC, NP, F), jnp.float32),
    mesh=_mesh,
    scratch_types=[
        pltpu.VMEM((EPW,), jnp.int32),
        pltpu.VMEM((K, F), jnp.float32),
        pltpu.VMEM_SHARED((NP, F), jnp.float32),
    ]
    + [pltpu.SemaphoreType.DMA] * U,
)
def _deg_kernel(dst_hbm, ones_hbm, zeros_hbm, out_hbm, dst_v, ones_v, acc,
                *sems):
    c = lax.axis_index("c")
    s = lax.axis_index("s")
    w = c * NS + s
    pltpu.sync_copy(dst_hbm.at[w], dst_v)
    pltpu.sync_copy(ones_hbm, ones_v)
    pltpu.sync_copy(zeros_hbm.at[pl.ds(s * RPT, RPT)], acc.at[pl.ds(s * RPT, RPT)])
    plsc.subcore_barrier()

    for b in range(U):
        pltpu.async_copy(ones_v, acc.at[dst_v.at[pl.ds(b * K, K)]], sems[b],
                         add=True)

    def body(gi, carry):
        g = gi * U
        for b in range(U):
            j = g + b
            pltpu.make_async_copy(ones_v, acc.at[dst_v.at[pl.ds(j * K, K)]],
                                  sems[b]).wait()
            pltpu.async_copy(ones_v,
                             acc.at[dst_v.at[pl.ds((j + U) * K, K)]],
                             sems[b], add=True)
        return carry

    lax.fori_loop(0, NCHUNK // U - 1, body, 0)
    for b in range(U):
        j = NCHUNK - U + b
        pltpu.make_async_copy(ones_v, acc.at[dst_v.at[pl.ds(j * K, K)]],
                              sems[b]).wait()
    plsc.subcore_barrier()
    pltpu.sync_copy(acc.at[pl.ds(s * RPT, RPT)], out_hbm.at[c, pl.ds(s * RPT, RPT)])


@functools.partial(
    pl.kernel,
    out_type=jax.ShapeDtypeStruct((NC, NP, F), jnp.float32),
    mesh=_mesh,
    scratch_types=[
        pltpu.VMEM((EPW,), jnp.int32),
        pltpu.VMEM((EPW,), jnp.int32),
        pltpu.VMEM((U, K, F), jnp.float32),
        pltpu.VMEM_SHARED((NP, F), jnp.float32),
    ]
    + [pltpu.SemaphoreType.DMA] * (2 * U),
)
def _agg_kernel(x_hbm, src_hbm, dst_hbm, zeros_hbm, out_hbm,
                src_v, dst_v, rows_v, acc, *sems):
    c = lax.axis_index("c")
    s = lax.axis_index("s")
    w = c * NS + s
    pltpu.sync_copy(src_hbm.at[w], src_v)
    pltpu.sync_copy(dst_hbm.at[w], dst_v)
    pltpu.sync_copy(zeros_hbm.at[pl.ds(s * RPT, RPT)], acc.at[pl.ds(s * RPT, RPT)])
    plsc.subcore_barrier()

    gsem = sems[:U]
    ssem = sems[U:]
    for b in range(U):
        pltpu.async_copy(x_hbm.at[src_v.at[pl.ds(b * K, K)]], rows_v.at[b],
                         gsem[b])

    def body(gi, carry):
        g = gi * U
        for b in range(U):
            j = g + b
            pltpu.make_async_copy(x_hbm.at[src_v.at[pl.ds(j * K, K)]],
                                  rows_v.at[b], gsem[b]).wait()
            pltpu.async_copy(rows_v.at[b], acc.at[dst_v.at[pl.ds(j * K, K)]],
                             ssem[b], add=True)
        for b in range(U):
            j = g + b
            pltpu.make_async_copy(rows_v.at[b],
                                  acc.at[dst_v.at[pl.ds(j * K, K)]],
                                  ssem[b]).wait()
            pltpu.async_copy(x_hbm.at[src_v.at[pl.ds((j + U) * K, K)]],
                             rows_v.at[b], gsem[b])
        return carry

    lax.fori_loop(0, NCHUNK // U - 1, body, 0)
    for b in range(U):
        j = NCHUNK - U + b
        pltpu.make_async_copy(x_hbm.at[src_v.at[pl.ds(j * K, K)]],
                              rows_v.at[b], gsem[b]).wait()
        pltpu.async_copy(rows_v.at[b], acc.at[dst_v.at[pl.ds(j * K, K)]],
                         ssem[b], add=True)
    for b in range(U):
        j = NCHUNK - U + b
        pltpu.make_async_copy(rows_v.at[b],
                              acc.at[dst_v.at[pl.ds(j * K, K)]], ssem[b]).wait()
    plsc.subcore_barrier()
    pltpu.sync_copy(acc.at[pl.ds(s * RPT, RPT)], out_hbm.at[c, pl.ds(s * RPT, RPT)])


def _gru(x, Wih, Whh, bih, bhh, d):
    gi = jnp.dot(x, Wih, preferred_element_type=jnp.float32) + bih
    gh = jnp.dot(x, Whh, preferred_element_type=jnp.float32) + bhh
    i_r, i_z, i_n = gi[:, :d], gi[:, d:2 * d], gi[:, 2 * d:]
    h_r, h_z, h_n = gh[:, :d], gh[:, d:2 * d], gh[:, 2 * d:]
    r = jax.nn.sigmoid(i_r + h_r)
    z = jax.nn.sigmoid(i_z + h_z)
    n = jnp.tanh(i_n + r * h_n)
    return (1.0 - z) * n + z * x


def _evolve_body(ns_ref, W1_ref, W2_ref, Wih1_ref, Whh1_ref, bih1_ref, bhh1_ref,
                 Wih2_ref, Whh2_ref, bih2_ref, bhh2_ref, Wm1_ref,
                 W1f_ref, Wc_ref):
    ns = ns_ref[0]

    def body(t, carry):
        W1, W2 = carry
        W1 = _gru(W1, Wih1_ref[...], Whh1_ref[...], bih1_ref[...], bhh1_ref[...], F)
        W2 = _gru(W2, Wih2_ref[...], Whh2_ref[...], bih2_ref[...], bhh2_ref[...], HID2)
        return (W1, W2)

    W1f, W2f = lax.fori_loop(0, ns, body, (W1_ref[...], W2_ref[...]))
    W1f_ref[...] = W1f
    Wc_ref[...] = jnp.dot(W2f, Wm1_ref[...], preferred_element_type=jnp.float32)


def _prep_body(d0_ref, d1_ref, x_ref, xp_ref, dinv_ref):
    deg = d0_ref[:, 0:1] + d1_ref[:, 0:1]
    dinv = lax.rsqrt(jnp.maximum(deg, 1.0))
    dinv_ref[...] = dinv
    xp_ref[...] = x_ref[...] * dinv


def _mid_body(z0_ref, z1_ref, dinv_ref, W1f_ref, hp_ref):
    dinv = dinv_ref[...]
    z = (z0_ref[...] + z1_ref[...]) * dinv
    h = jnp.dot(z, W1f_ref[...], preferred_element_type=jnp.float32)
    h = jnp.where(h >= 0.0, h, SLOPE * h)
    hp_ref[...] = h * dinv


def _final_body(u0_ref, u1_ref, dinv_ref, Wc_ref, bm1_ref, Wm2_ref, bm2_ref, out_ref):
    u = (u0_ref[...] + u1_ref[...]) * dinv_ref[...]
    t = jnp.dot(u, Wc_ref[...], preferred_element_type=jnp.float32) + bm1_ref[...]
    t = jnp.maximum(t, 0.0)
    out_ref[...] = jnp.dot(t, Wm2_ref[...], preferred_element_type=jnp.float32) + bm2_ref[...]


def kernel(feat_list, edge_index, n_step, W1, W2, Wih1, Whh1, bih1, bhh1,
           Wih2, Whh2, bih2, bhh2, Wm1, bm1, Wm2, bm2):
    src = edge_index[0].reshape(NW, EPW)
    dst = edge_index[1].reshape(NW, EPW)
    x_last = lax.dynamic_index_in_dim(feat_list, n_step - 1, 0, keepdims=False)
    x_last = jnp.pad(x_last, ((0, NP - N), (0, 0)))

    ones_d = jnp.ones((K, F), jnp.float32)
    zeros_f = jnp.zeros((NP, F), jnp.float32)

    deg_parts = _deg_kernel(dst, ones_d, zeros_f)

    W1f, Wc = pl.pallas_call(
        _evolve_body,
        out_shape=[
            jax.ShapeDtypeStruct((F, F), jnp.float32),
            jax.ShapeDtypeStruct((F, CF), jnp.float32),
        ],
        in_specs=[pl.BlockSpec(memory_space=pltpu.SMEM)] + [pl.BlockSpec()] * 11,
    )(jnp.asarray(n_step, jnp.int32).reshape(1), W1, W2,
      Wih1, Whh1, bih1.reshape(1, -1), bhh1.reshape(1, -1),
      Wih2, Whh2, bih2.reshape(1, -1), bhh2.reshape(1, -1), Wm1)

    xp, dinv = pl.pallas_call(
        _prep_body,
        out_shape=[
            jax.ShapeDtypeStruct((NP, F), jnp.float32),
            jax.ShapeDtypeStruct((NP, 1), jnp.float32),
        ],
    )(deg_parts[0], deg_parts[1], x_last)

    z_parts = _agg_kernel(xp, src, dst, zeros_f)

    hp = pl.pallas_call(
        _mid_body,
        out_shape=jax.ShapeDtypeStruct((NP, F), jnp.float32),
    )(z_parts[0], z_parts[1], dinv, W1f)

    u_parts = _agg_kernel(hp, src, dst, zeros_f)

    out = pl.pallas_call(
        _final_body,
        out_shape=jax.ShapeDtypeStruct((NP, OF), jnp.float32),
    )(u_parts[0], u_parts[1], dinv, Wc, bm1.reshape(1, -1), Wm2, bm2.reshape(1, -1))
    return out[:N]
